# baseline (speedup 1.0000x reference)
import functools

import jax
import jax.numpy as jnp
from jax import lax
from jax.experimental import pallas as pl
from jax.experimental.pallas import tpu as pltpu
from jax.experimental.pallas import tpu_sc as plsc

N_NODES = 10000
N_EDGES = 320000
D = 128

NC = 2
NS = 16
NW = NC * NS
CHUNK = 128
RPW = 80
RPH = RPW // 2
EPW = RPW * CHUNK
EPAD = NW * EPW
EXTRA = 0
NPAD = 10240
ROWS_PER_TILE = NPAD // NS


def _sc_aggregate(x, src, dst, zrows):
    mesh = plsc.VectorSubcoreMesh(core_axis_name="c", subcore_axis_name="s")

    @functools.partial(
        pl.kernel,
        mesh=mesh,
        out_type=jax.ShapeDtypeStruct((NC, NPAD, D), jnp.float32),
        scratch_types=[
            pltpu.VMEM((RPH, CHUNK), jnp.int32),
            pltpu.VMEM((RPH, CHUNK), jnp.int32),
            pltpu.VMEM((CHUNK, D), jnp.float32),
            pltpu.VMEM((CHUNK, D), jnp.float32),
            pltpu.VMEM_SHARED((NPAD, D), jnp.float32),
            pltpu.SemaphoreType.DMA,
            pltpu.SemaphoreType.DMA,
            pltpu.SemaphoreType.DMA,
            pltpu.SemaphoreType.DMA,
        ],
    )
    def agg_kernel(x_hbm, src_hbm, dst_hbm, z_hbm, out_hbm,
                   srcv, dstv, rows_a, rows_b, agg_sh,
                   sem_a, sem_b, sem_sa, sem_sb):
        cid = lax.axis_index("c")
        sid = lax.axis_index("s")
        wid = sid * NC + cid

        pltpu.sync_copy(z_hbm,
                        agg_sh.at[pl.ds(sid * ROWS_PER_TILE, ROWS_PER_TILE)])
        plsc.subcore_barrier()

        rbase = wid * RPW
        for h in range(RPW // RPH):
            hbase = rbase + h * RPH
            pltpu.sync_copy(src_hbm.at[pl.ds(hbase, RPH)], srcv)
            pltpu.sync_copy(dst_hbm.at[pl.ds(hbase, RPH)], dstv)

            pltpu.async_copy(x_hbm.at[srcv.at[0]], rows_a, sem_a)
            pltpu.async_copy(x_hbm.at[srcv.at[1]], rows_b, sem_b)

            def body(k, carry):
                j0 = 2 * k
                j1 = j0 + 1
                pltpu.make_async_copy(
                    x_hbm.at[srcv.at[j0]], rows_a, sem_a).wait()
                pltpu.async_copy(rows_a, agg_sh.at[dstv.at[j0]], sem_sa,
                                 add=True)
                pltpu.make_async_copy(
                    x_hbm.at[srcv.at[j1]], rows_b, sem_b).wait()
                pltpu.async_copy(rows_b, agg_sh.at[dstv.at[j1]], sem_sb,
                                 add=True)
                pltpu.make_async_copy(
                    rows_a, agg_sh.at[dstv.at[j0]], sem_sa).wait()

                @pl.when(k < RPH // 2 - 1)
                def _():
                    pltpu.async_copy(x_hbm.at[srcv.at[j0 + 2]], rows_a, sem_a)

                pltpu.make_async_copy(
                    rows_b, agg_sh.at[dstv.at[j1]], sem_sb).wait()

                @pl.when(k < RPH // 2 - 1)
                def _():
                    pltpu.async_copy(x_hbm.at[srcv.at[j1 + 2]], rows_b, sem_b)

                return carry

            lax.fori_loop(0, RPH // 2, body, 0)

        plsc.subcore_barrier()
        pltpu.sync_copy(
            agg_sh.at[pl.ds(sid * ROWS_PER_TILE, ROWS_PER_TILE)],
            out_hbm.at[cid, pl.ds(sid * ROWS_PER_TILE, ROWS_PER_TILE)])

    return agg_kernel(x, src, dst, zrows)


BN = 2000


def _tc_finish(parts, x, W):
    def body(p_ref, x_ref, w_ref, o_ref):
        agg = p_ref[0] + p_ref[1]
        h = lax.dot_general(agg, w_ref[...], (((1,), (1,)), ((), ())),
                            preferred_element_type=jnp.float32)
        o_ref[...] = jnp.maximum(h, 0.0) + x_ref[...]

    return pl.pallas_call(
        body,
        grid=(N_NODES // BN,),
        in_specs=[
            pl.BlockSpec((NC, BN, D), lambda i: (0, i, 0)),
            pl.BlockSpec((BN, D), lambda i: (i, 0)),
            pl.BlockSpec((D, D), lambda i: (0, 0)),
        ],
        out_specs=pl.BlockSpec((BN, D), lambda i: (i, 0)),
        out_shape=jax.ShapeDtypeStruct((N_NODES, D), jnp.float32),
    )(parts, x, W)


def kernel(x, edge_index, W):
    src = edge_index[0].astype(jnp.int32)
    dst = edge_index[1].astype(jnp.int32)
    pad = EPAD + EXTRA - N_EDGES
    arange_pad = jnp.arange(pad, dtype=jnp.int32)
    src_p = jnp.concatenate(
        [src, N_NODES + arange_pad % (NPAD - N_NODES)]).reshape(-1, CHUNK)
    dst_p = jnp.concatenate([dst, arange_pad % NPAD]).reshape(-1, CHUNK)
    x_g = jnp.concatenate(
        [x, jnp.zeros((NPAD - N_NODES, D), jnp.float32)])
    zrows = jnp.zeros((ROWS_PER_TILE, D), jnp.float32)
    parts = _sc_aggregate(x_g, src_p, dst_p, zrows)
    return _tc_finish(parts, x, W)


# speedup vs baseline: 1.2628x; 1.2628x over previous
import functools

import jax
import jax.numpy as jnp
from jax import lax
from jax.experimental import pallas as pl
from jax.experimental.pallas import tpu as pltpu
from jax.experimental.pallas import tpu_sc as plsc

N_NODES = 10000
N_EDGES = 320000
D = 128

NC = 2
NS = 16
NW = NC * NS
CHUNK = 128
CROWS = N_EDGES // CHUNK
CROWS_P = 2504
RPW = 80
RPH = RPW // 2
RLAST = CROWS - 31 * RPW
RLOAD = 24
NPAD = 10240
ROWS_PER_TILE = NPAD // NS


def _sc_aggregate(x, src, dst, zrows):
    mesh = plsc.VectorSubcoreMesh(core_axis_name="c", subcore_axis_name="s")

    @functools.partial(
        pl.kernel,
        mesh=mesh,
        out_type=jax.ShapeDtypeStruct((NC, NPAD, D), jnp.float32),
        scratch_types=[
            pltpu.VMEM((RPH, CHUNK), jnp.int32),
            pltpu.VMEM((RPH, CHUNK), jnp.int32),
            pltpu.VMEM((CHUNK, D), jnp.float32),
            pltpu.VMEM((CHUNK, D), jnp.float32),
            pltpu.VMEM_SHARED((NPAD, D), jnp.float32),
            pltpu.SemaphoreType.DMA,
            pltpu.SemaphoreType.DMA,
        ],
    )
    def agg_kernel(x_hbm, src_hbm, dst_hbm, z_hbm, out_hbm,
                   srcv, dstv, rows_a, rows_b, agg_sh, sem_a, sem_b):
        cid = lax.axis_index("c")
        sid = lax.axis_index("s")
        wid = sid * NC + cid

        pltpu.sync_copy(z_hbm,
                        agg_sh.at[pl.ds(sid * ROWS_PER_TILE, ROWS_PER_TILE)])
        plsc.subcore_barrier()

        def run_slab(hbase, nload, n):
            pltpu.sync_copy(src_hbm.at[pl.ds(hbase, nload)],
                            srcv.at[pl.ds(0, nload)])
            pltpu.sync_copy(dst_hbm.at[pl.ds(hbase, nload)],
                            dstv.at[pl.ds(0, nload)])

            pltpu.async_copy(x_hbm.at[srcv.at[0]], rows_a, sem_a)

            def body(k, carry):
                j0 = 2 * k
                j1 = j0 + 1
                pltpu.async_copy(x_hbm.at[srcv.at[j1]], rows_b, sem_b)
                pltpu.make_async_copy(
                    x_hbm.at[srcv.at[j0]], rows_a, sem_a).wait()
                pltpu.sync_copy(rows_a, agg_sh.at[dstv.at[j0]], add=True)

                @pl.when(k < n // 2 - 1)
                def _():
                    pltpu.async_copy(x_hbm.at[srcv.at[j0 + 2]], rows_a, sem_a)

                pltpu.make_async_copy(
                    x_hbm.at[srcv.at[j1]], rows_b, sem_b).wait()
                pltpu.sync_copy(rows_b, agg_sh.at[dstv.at[j1]], add=True)
                return carry

            lax.fori_loop(0, n // 2, body, 0)

        rbase = wid * RPW

        @pl.when(wid < NW - 1)
        def _():
            run_slab(rbase, RPH, RPH)
            run_slab(rbase + RPH, RPH, RPH)

        @pl.when(wid == NW - 1)
        def _():
            run_slab(rbase, RLOAD, RLAST)

        plsc.subcore_barrier()
        pltpu.sync_copy(
            agg_sh.at[pl.ds(sid * ROWS_PER_TILE, ROWS_PER_TILE)],
            out_hbm.at[cid, pl.ds(sid * ROWS_PER_TILE, ROWS_PER_TILE)])

    return agg_kernel(x, src, dst, zrows)


BN = 2000


def _tc_finish(parts, x, W):
    def body(p_ref, x_ref, w_ref, o_ref):
        agg = p_ref[0] + p_ref[1]
        h = lax.dot_general(agg, w_ref[...], (((1,), (1,)), ((), ())),
                            preferred_element_type=jnp.float32)
        o_ref[...] = jnp.maximum(h, 0.0) + x_ref[...]

    return pl.pallas_call(
        body,
        grid=(N_NODES // BN,),
        in_specs=[
            pl.BlockSpec((NC, BN, D), lambda i: (0, i, 0)),
            pl.BlockSpec((BN, D), lambda i: (i, 0)),
            pl.BlockSpec((D, D), lambda i: (0, 0)),
        ],
        out_specs=pl.BlockSpec((BN, D), lambda i: (i, 0)),
        out_shape=jax.ShapeDtypeStruct((N_NODES, D), jnp.float32),
    )(parts, x, W)


def kernel(x, edge_index, W):
    src = edge_index[0].astype(jnp.int32)
    dst = edge_index[1].astype(jnp.int32)
    pad = (CROWS_P - CROWS) * CHUNK
    src_p = jnp.concatenate(
        [src, jnp.zeros((pad,), jnp.int32)]).reshape(CROWS_P, CHUNK)
    dst_p = jnp.concatenate(
        [dst, jnp.zeros((pad,), jnp.int32)]).reshape(CROWS_P, CHUNK)
    zrows = jnp.zeros((ROWS_PER_TILE, D), jnp.float32)
    parts = _sc_aggregate(x, src_p, dst_p, zrows)
    return _tc_finish(parts, x, W)


# speedup vs baseline: 1.3673x; 1.0828x over previous
import functools

import jax
import jax.numpy as jnp
from jax import lax
from jax.experimental import pallas as pl
from jax.experimental.pallas import tpu as pltpu
from jax.experimental.pallas import tpu_sc as plsc

N_NODES = 10000
N_EDGES = 320000
D = 128

NC = 2
NS = 16
NW = NC * NS
CHUNK = 128
CROWS = N_EDGES // CHUNK
CROWS_P = 2504
RPW = 80
RPH = RPW // 2
RLAST = CROWS - 31 * RPW
RLOAD = 24
NPAD = 10240
ROWS_PER_TILE = NPAD // NS


def _sc_aggregate(x, edges, zrows):
    mesh = plsc.VectorSubcoreMesh(core_axis_name="c", subcore_axis_name="s")

    @functools.partial(
        pl.kernel,
        mesh=mesh,
        out_type=jax.ShapeDtypeStruct((NC, NPAD, D), jnp.float32),
        scratch_types=[
            pltpu.VMEM((RPH, CHUNK), jnp.int32),
            pltpu.VMEM((RPH, CHUNK), jnp.int32),
            pltpu.VMEM((CHUNK, D), jnp.float32),
            pltpu.VMEM((CHUNK, D), jnp.float32),
            pltpu.VMEM_SHARED((NPAD, D), jnp.float32),
            pltpu.SemaphoreType.DMA,
            pltpu.SemaphoreType.DMA,
        ],
    )
    def agg_kernel(x_hbm, edges_hbm, z_hbm, out_hbm,
                   srcv, dstv, rows_a, rows_b, agg_sh, sem_a, sem_b):
        cid = lax.axis_index("c")
        sid = lax.axis_index("s")
        wid = sid * NC + cid

        pltpu.sync_copy(z_hbm,
                        agg_sh.at[pl.ds(sid * ROWS_PER_TILE, ROWS_PER_TILE)])
        plsc.subcore_barrier()

        def run_slab(hbase, nload, n):
            pltpu.sync_copy(edges_hbm.at[0, pl.ds(hbase, nload)],
                            srcv.at[pl.ds(0, nload)])
            pltpu.sync_copy(edges_hbm.at[1, pl.ds(hbase, nload)],
                            dstv.at[pl.ds(0, nload)])

            pltpu.async_copy(x_hbm.at[srcv.at[0]], rows_a, sem_a)

            def body(k, carry):
                j0 = 2 * k
                j1 = j0 + 1
                pltpu.async_copy(x_hbm.at[srcv.at[j1]], rows_b, sem_b)
                pltpu.make_async_copy(
                    x_hbm.at[srcv.at[j0]], rows_a, sem_a).wait()
                pltpu.sync_copy(rows_a, agg_sh.at[dstv.at[j0]], add=True)

                @pl.when(k < n // 2 - 1)
                def _():
                    pltpu.async_copy(x_hbm.at[srcv.at[j0 + 2]], rows_a, sem_a)

                pltpu.make_async_copy(
                    x_hbm.at[srcv.at[j1]], rows_b, sem_b).wait()
                pltpu.sync_copy(rows_b, agg_sh.at[dstv.at[j1]], add=True)
                return carry

            lax.fori_loop(0, n // 2, body, 0)

        rbase = wid * RPW

        @pl.when(wid < NW - 1)
        def _():
            run_slab(rbase, RPH, RPH)
            run_slab(rbase + RPH, RPH, RPH)

        @pl.when(wid == NW - 1)
        def _():
            run_slab(rbase, RLOAD, RLAST)

        plsc.subcore_barrier()
        pltpu.sync_copy(
            agg_sh.at[pl.ds(sid * ROWS_PER_TILE, ROWS_PER_TILE)],
            out_hbm.at[cid, pl.ds(sid * ROWS_PER_TILE, ROWS_PER_TILE)])

    return agg_kernel(x, edges, zrows)


BN = 2000


def _tc_finish(parts, x, W):
    def body(p_ref, x_ref, w_ref, o_ref):
        agg = p_ref[0] + p_ref[1]
        h = lax.dot_general(agg, w_ref[...], (((1,), (1,)), ((), ())),
                            preferred_element_type=jnp.float32)
        o_ref[...] = jnp.maximum(h, 0.0) + x_ref[...]

    return pl.pallas_call(
        body,
        grid=(N_NODES // BN,),
        in_specs=[
            pl.BlockSpec((NC, BN, D), lambda i: (0, i, 0)),
            pl.BlockSpec((BN, D), lambda i: (i, 0)),
            pl.BlockSpec((D, D), lambda i: (0, 0)),
        ],
        out_specs=pl.BlockSpec((BN, D), lambda i: (i, 0)),
        out_shape=jax.ShapeDtypeStruct((N_NODES, D), jnp.float32),
    )(parts, x, W)


def kernel(x, edge_index, W):
    pad = (CROWS_P - CROWS) * CHUNK
    edges = jnp.concatenate(
        [edge_index.astype(jnp.int32),
         jnp.zeros((2, pad), jnp.int32)], axis=1).reshape(2, CROWS_P, CHUNK)
    zrows = jnp.zeros((ROWS_PER_TILE, D), jnp.float32)
    parts = _sc_aggregate(x, edges, zrows)
    return _tc_finish(parts, x, W)


# speedup vs baseline: 1.4039x; 1.0268x over previous
import functools

import jax
import jax.numpy as jnp
from jax import lax
from jax.experimental import pallas as pl
from jax.experimental.pallas import tpu as pltpu
from jax.experimental.pallas import tpu_sc as plsc

N_NODES = 10000
N_EDGES = 320000
D = 128

NC = 2
NS = 16
NW = NC * NS
CHUNK = 128
CROWS = N_EDGES // CHUNK
CROWS_P = 2504
RPW = 80
RPH = RPW // 2
RLAST = CROWS - 31 * RPW
RLOAD = 24
NPAD = 10240
ROWS_PER_TILE = NPAD // NS


def _sc_aggregate(x, edges):
    mesh = plsc.VectorSubcoreMesh(core_axis_name="c", subcore_axis_name="s")

    @functools.partial(
        pl.kernel,
        mesh=mesh,
        out_type=jax.ShapeDtypeStruct((NC, NPAD, D), jnp.float32),
        scratch_types=[
            pltpu.VMEM((RPH, CHUNK), jnp.int32),
            pltpu.VMEM((RPH, CHUNK), jnp.int32),
            pltpu.VMEM((CHUNK, D), jnp.float32),
            pltpu.VMEM((CHUNK, D), jnp.float32),
            pltpu.VMEM_SHARED((NPAD, D), jnp.float32),
            pltpu.SemaphoreType.DMA,
            pltpu.SemaphoreType.DMA,
        ],
    )
    def agg_kernel(x_hbm, edges_hbm, out_hbm,
                   srcv, dstv, rows_a, rows_b, agg_sh, sem_a, sem_b):
        cid = lax.axis_index("c")
        sid = lax.axis_index("s")
        wid = sid * NC + cid

        zv = jnp.zeros((16,), jnp.float32)

        def zrow(i, carry):
            for c in range(D // 16):
                rows_a[i, pl.ds(c * 16, 16)] = zv
            return carry

        lax.fori_loop(0, CHUNK, zrow, 0)
        for q in range(ROWS_PER_TILE // CHUNK):
            pltpu.sync_copy(
                rows_a,
                agg_sh.at[pl.ds(sid * ROWS_PER_TILE + q * CHUNK, CHUNK)])
        plsc.subcore_barrier()

        def run_slab(hbase, nload, n):
            pltpu.sync_copy(edges_hbm.at[0, pl.ds(hbase, nload)],
                            srcv.at[pl.ds(0, nload)])
            pltpu.sync_copy(edges_hbm.at[1, pl.ds(hbase, nload)],
                            dstv.at[pl.ds(0, nload)])

            pltpu.async_copy(x_hbm.at[srcv.at[0]], rows_a, sem_a)

            def body(k, carry):
                j0 = 2 * k
                j1 = j0 + 1
                pltpu.async_copy(x_hbm.at[srcv.at[j1]], rows_b, sem_b)
                pltpu.make_async_copy(
                    x_hbm.at[srcv.at[j0]], rows_a, sem_a).wait()
                pltpu.sync_copy(rows_a, agg_sh.at[dstv.at[j0]], add=True)

                @pl.when(k < n // 2 - 1)
                def _():
                    pltpu.async_copy(x_hbm.at[srcv.at[j0 + 2]], rows_a, sem_a)

                pltpu.make_async_copy(
                    x_hbm.at[srcv.at[j1]], rows_b, sem_b).wait()
                pltpu.sync_copy(rows_b, agg_sh.at[dstv.at[j1]], add=True)
                return carry

            lax.fori_loop(0, n // 2, body, 0)

        rbase = wid * RPW

        @pl.when(wid < NW - 1)
        def _():
            run_slab(rbase, RPH, RPH)
            run_slab(rbase + RPH, RPH, RPH)

        @pl.when(wid == NW - 1)
        def _():
            run_slab(rbase, RLOAD, RLAST)

        plsc.subcore_barrier()
        pltpu.sync_copy(
            agg_sh.at[pl.ds(sid * ROWS_PER_TILE, ROWS_PER_TILE)],
            out_hbm.at[cid, pl.ds(sid * ROWS_PER_TILE, ROWS_PER_TILE)])

    return agg_kernel(x, edges)


BN = 2000


def _tc_finish(parts, x, W):
    def body(p_ref, x_ref, w_ref, o_ref):
        agg = p_ref[0] + p_ref[1]
        h = lax.dot_general(agg, w_ref[...], (((1,), (1,)), ((), ())),
                            preferred_element_type=jnp.float32)
        o_ref[...] = jnp.maximum(h, 0.0) + x_ref[...]

    return pl.pallas_call(
        body,
        grid=(N_NODES // BN,),
        in_specs=[
            pl.BlockSpec((NC, BN, D), lambda i: (0, i, 0)),
            pl.BlockSpec((BN, D), lambda i: (i, 0)),
            pl.BlockSpec((D, D), lambda i: (0, 0)),
        ],
        out_specs=pl.BlockSpec((BN, D), lambda i: (i, 0)),
        out_shape=jax.ShapeDtypeStruct((N_NODES, D), jnp.float32),
    )(parts, x, W)


def kernel(x, edge_index, W):
    pad = (CROWS_P - CROWS) * CHUNK
    edges = jnp.concatenate(
        [edge_index.astype(jnp.int32),
         jnp.zeros((2, pad), jnp.int32)], axis=1).reshape(2, CROWS_P, CHUNK)
    parts = _sc_aggregate(x, edges)
    return _tc_finish(parts, x, W)


# speedup vs baseline: 1.4320x; 1.0200x over previous
import functools

import jax
import jax.numpy as jnp
from jax import lax
from jax.experimental import pallas as pl
from jax.experimental.pallas import tpu as pltpu
from jax.experimental.pallas import tpu_sc as plsc

N_NODES = 10000
N_EDGES = 320000
D = 128

NC = 2
NS = 16
NW = NC * NS
CHUNK = 128
CROWS = N_EDGES // CHUNK
CROWS_P = 2504
RPW = 80
RPH = RPW // 2
RLAST = CROWS - 31 * RPW
RLOAD = 24
NPAD = 10240
ROWS_PER_TILE = NPAD // NS


def _sc_aggregate(x, edges):
    mesh = plsc.VectorSubcoreMesh(core_axis_name="c", subcore_axis_name="s")

    @functools.partial(
        pl.kernel,
        mesh=mesh,
        out_type=jax.ShapeDtypeStruct((NC, NPAD, D), jnp.float32),
        scratch_types=[
            pltpu.VMEM((RPH, CHUNK), jnp.int32),
            pltpu.VMEM((RPH, CHUNK), jnp.int32),
            pltpu.VMEM((CHUNK, D), jnp.float32),
            pltpu.VMEM((CHUNK, D), jnp.float32),
            pltpu.VMEM_SHARED((NPAD, D), jnp.float32),
            pltpu.SemaphoreType.DMA,
            pltpu.SemaphoreType.DMA,
        ],
    )
    def agg_kernel(x_hbm, edges_hbm, out_hbm,
                   srcv, dstv, rows_a, rows_b, agg_sh, sem_a, sem_b):
        cid = lax.axis_index("c")
        sid = lax.axis_index("s")
        wid = sid * NC + cid

        zv = jnp.zeros((16,), jnp.float32)

        def zrow(i, carry):
            for c in range(D // 16):
                rows_a[i, pl.ds(c * 16, 16)] = zv
            return carry

        lax.fori_loop(0, CHUNK, zrow, 0)
        for q in range(ROWS_PER_TILE // CHUNK):
            pltpu.sync_copy(
                rows_a,
                agg_sh.at[pl.ds(sid * ROWS_PER_TILE + q * CHUNK, CHUNK)])
        plsc.subcore_barrier()

        def run_slab(hbase, nload, n):
            pltpu.sync_copy(edges_hbm.at[0, pl.ds(hbase, nload)],
                            srcv.at[pl.ds(0, nload)])
            pltpu.async_copy(x_hbm.at[srcv.at[0]], rows_a, sem_a)
            pltpu.sync_copy(edges_hbm.at[1, pl.ds(hbase, nload)],
                            dstv.at[pl.ds(0, nload)])

            def body(k, carry):
                j0 = 2 * k
                j1 = j0 + 1
                pltpu.async_copy(x_hbm.at[srcv.at[j1]], rows_b, sem_b)
                pltpu.make_async_copy(
                    x_hbm.at[srcv.at[j0]], rows_a, sem_a).wait()
                pltpu.sync_copy(rows_a, agg_sh.at[dstv.at[j0]], add=True)
                pltpu.async_copy(x_hbm.at[srcv.at[j0 + 2]], rows_a, sem_a)
                pltpu.make_async_copy(
                    x_hbm.at[srcv.at[j1]], rows_b, sem_b).wait()
                pltpu.sync_copy(rows_b, agg_sh.at[dstv.at[j1]], add=True)
                return carry

            lax.fori_loop(0, n // 2 - 1, body, 0)

            pltpu.async_copy(x_hbm.at[srcv.at[n - 1]], rows_b, sem_b)
            pltpu.make_async_copy(x_hbm.at[srcv.at[n - 2]], rows_a,
                                  sem_a).wait()
            pltpu.sync_copy(rows_a, agg_sh.at[dstv.at[n - 2]], add=True)
            pltpu.make_async_copy(x_hbm.at[srcv.at[n - 1]], rows_b,
                                  sem_b).wait()
            pltpu.sync_copy(rows_b, agg_sh.at[dstv.at[n - 1]], add=True)

        rbase = wid * RPW

        @pl.when(wid < NW - 1)
        def _():
            run_slab(rbase, RPH, RPH)
            run_slab(rbase + RPH, RPH, RPH)

        @pl.when(wid == NW - 1)
        def _():
            run_slab(rbase, RLOAD, RLAST)

        plsc.subcore_barrier()
        pltpu.sync_copy(
            agg_sh.at[pl.ds(sid * ROWS_PER_TILE, ROWS_PER_TILE)],
            out_hbm.at[cid, pl.ds(sid * ROWS_PER_TILE, ROWS_PER_TILE)])

    return agg_kernel(x, edges)


BN = 2000


def _tc_finish(parts, x, W):
    def body(p_ref, x_ref, w_ref, o_ref):
        agg = p_ref[0] + p_ref[1]
        h = lax.dot_general(agg, w_ref[...], (((1,), (1,)), ((), ())),
                            preferred_element_type=jnp.float32)
        o_ref[...] = jnp.maximum(h, 0.0) + x_ref[...]

    return pl.pallas_call(
        body,
        grid=(N_NODES // BN,),
        in_specs=[
            pl.BlockSpec((NC, BN, D), lambda i: (0, i, 0)),
            pl.BlockSpec((BN, D), lambda i: (i, 0)),
            pl.BlockSpec((D, D), lambda i: (0, 0)),
        ],
        out_specs=pl.BlockSpec((BN, D), lambda i: (i, 0)),
        out_shape=jax.ShapeDtypeStruct((N_NODES, D), jnp.float32),
    )(parts, x, W)


def kernel(x, edge_index, W):
    pad = (CROWS_P - CROWS) * CHUNK
    edges = jnp.concatenate(
        [edge_index.astype(jnp.int32),
         jnp.zeros((2, pad), jnp.int32)], axis=1).reshape(2, CROWS_P, CHUNK)
    parts = _sc_aggregate(x, edges)
    return _tc_finish(parts, x, W)
